# baseline (speedup 1.0000x reference)
import jax
import jax.numpy as jnp
from jax.experimental import pallas as pl
from jax.experimental.pallas import tpu as pltpu

_T = 0.07
_BM = 16
_NSLOT = 6


def _mm_kernel(x_ref, mt_hbm, o_hbm, mt_vmem, obuf, insem, outsems):
    i = pl.program_id(0)
    nsteps = pl.num_programs(0)
    n = mt_vmem.shape[1]

    @pl.when(i == 0)
    def _load_mt():
        cp = pltpu.make_async_copy(mt_hbm, mt_vmem, insem)
        cp.start()
        cp.wait()

    slot = jax.lax.rem(i, _NSLOT)

    @pl.when(i >= _NSLOT)
    def _free_slot():
        pltpu.make_async_copy(
            obuf.at[slot], o_hbm.at[pl.ds(0, _BM), :], outsems.at[slot]
        ).wait()

    xs = x_ref[...] * (1.0 / _T)
    obuf[slot, :, :] = jax.lax.dot_general(
        xs, mt_vmem[...],
        dimension_numbers=(((1,), (0,)), ((), ())),
        preferred_element_type=jnp.float32)

    pltpu.make_async_copy(
        obuf.at[slot], o_hbm.at[pl.ds(i * _BM, _BM), :], outsems.at[slot]
    ).start()

    @pl.when(i == nsteps - 1)
    def _drain():
        for k in range(_NSLOT):
            pltpu.make_async_copy(
                obuf.at[k], o_hbm.at[pl.ds(0, _BM), :], outsems.at[k]
            ).wait()


def kernel(x, y, memory):
    M, K = x.shape
    N = memory.shape[0]
    mt = memory.T
    return pl.pallas_call(
        _mm_kernel,
        grid=(M // _BM,),
        in_specs=[
            pl.BlockSpec((_BM, K), lambda i: (i, 0)),
            pl.BlockSpec(memory_space=pltpu.HBM),
        ],
        out_specs=pl.BlockSpec(memory_space=pltpu.HBM),
        out_shape=jax.ShapeDtypeStruct((M, N), jnp.float32),
        scratch_shapes=[
            pltpu.VMEM((K, N), jnp.float32),
            pltpu.VMEM((_NSLOT, _BM, N), jnp.float32),
            pltpu.SemaphoreType.DMA,
            pltpu.SemaphoreType.DMA((_NSLOT,)),
        ],
    )(x, mt)
